# Initial kernel scaffold; baseline (speedup 1.0000x reference)
import functools

import jax
import jax.numpy as jnp
from jax import lax
from jax.experimental import pallas as pl
from jax.experimental.pallas import tpu as pltpu
from jax.experimental.pallas import tpu_sc as plsc

NC = 2
NS = 16
NW = NC * NS
CH = 80


def _silu(v):
    return v * jax.nn.sigmoid(v)


def _prep_body(h_ref, g_ref, b_ref, wa_ref, wb_ref, hb_ref, a_ref, bb_ref):
    h = h_ref[...]
    mean = jnp.mean(h, axis=0, keepdims=True)
    var = jnp.mean((h - mean) ** 2, axis=0, keepdims=True)
    hb = g_ref[...] * (h - mean) / jnp.sqrt(var + 1e-5) + b_ref[...]
    hb_ref[...] = hb
    a_ref[...] = jnp.dot(hb, wa_ref[...], preferred_element_type=jnp.float32)
    bb_ref[...] = jnp.dot(hb, wb_ref[...], preferred_element_type=jnp.float32)


def _gather_body(n_nodes, epw, a_hbm, b_hbm, xc_hbm, es_hbm, ed_hbm,
                 as_out, bd_out, d2_out,
                 x0, x1, x2, idx_s, idx_d, asb, bdb, d2b, sem):
    c = lax.axis_index("c")
    s = lax.axis_index("s")
    base = (s * NC + c) * epw
    pltpu.sync_copy(xc_hbm.at[0], x0)
    pltpu.sync_copy(xc_hbm.at[1], x1)
    pltpu.sync_copy(xc_hbm.at[2], x2)

    def chunk(j, carry):
        off = base + j * CH
        pltpu.sync_copy(es_hbm.at[pl.ds(off, CH)], idx_s)
        pltpu.sync_copy(ed_hbm.at[pl.ds(off, CH)], idx_d)
        cp_a = pltpu.async_copy(a_hbm.at[idx_s], asb, sem)
        cp_b = pltpu.async_copy(b_hbm.at[idx_d], bdb, sem)
        for i in range(CH // 16):
            si = idx_s[pl.ds(i * 16, 16)]
            di = idx_d[pl.ds(i * 16, 16)]
            dx = plsc.load_gather(x0, [si]) - plsc.load_gather(x0, [di])
            dy = plsc.load_gather(x1, [si]) - plsc.load_gather(x1, [di])
            dz = plsc.load_gather(x2, [si]) - plsc.load_gather(x2, [di])
            d2b[pl.ds(i * 16, 16)] = dx * dx + dy * dy + dz * dz
        cp_a.wait()
        cp_b.wait()
        pltpu.sync_copy(asb, as_out.at[pl.ds(off, CH)])
        pltpu.sync_copy(bdb, bd_out.at[pl.ds(off, CH)])
        pltpu.sync_copy(d2b, d2_out.at[pl.ds(off, CH)])
        return carry

    lax.fori_loop(0, epw // CH, chunk, 0)


def _edge_body(as_ref, bd_ref, d2_ref, w256_ref, be1_ref, we2_ref, be2_ref,
               wit_ref, bi_ref, out_ref):
    dist = jnp.sqrt(d2_ref[...])
    pre1 = as_ref[...] + bd_ref[...] + dist * w256_ref[...] + be1_ref[...]
    u = _silu(pre1)
    v = jnp.dot(u, we2_ref[...], preferred_element_type=jnp.float32)
    v = _silu(v + be2_ref[...])
    logit = jnp.sum(v * wit_ref[...], axis=1, keepdims=True) + bi_ref[0]
    out_ref[...] = jax.nn.sigmoid(logit) * v


def _scatter_body(n_nodes, epw, wm_hbm, es_hbm, zeros_hbm, out_hbm,
                  acc, rows, idx, sem):
    c = lax.axis_index("c")
    s = lax.axis_index("s")
    npt = (n_nodes // NS) // 8 * 8
    rem = n_nodes - npt * NS
    pltpu.sync_copy(zeros_hbm.at[pl.ds(s * npt, npt)],
                    acc.at[pl.ds(s * npt, npt)])

    @pl.when(s == 0)
    def _():
        pltpu.sync_copy(zeros_hbm.at[pl.ds(npt * NS, rem)],
                        acc.at[pl.ds(npt * NS, rem)])

    plsc.subcore_barrier()
    base = (s * NC + c) * epw

    def chunk(j, carry):
        off = base + j * CH
        pltpu.sync_copy(es_hbm.at[pl.ds(off, CH)], idx)
        pltpu.sync_copy(wm_hbm.at[pl.ds(off, CH)], rows)
        pltpu.sync_copy(rows, acc.at[idx], add=True)
        return carry

    lax.fori_loop(0, epw // CH, chunk, 0)
    plsc.subcore_barrier()
    obase = c * n_nodes + s * npt
    pltpu.sync_copy(acc.at[pl.ds(s * npt, npt)], out_hbm.at[pl.ds(obase, npt)])

    @pl.when(s == 0)
    def _():
        pltpu.sync_copy(acc.at[pl.ds(npt * NS, rem)],
                        out_hbm.at[pl.ds(c * n_nodes + npt * NS, rem)])


def _final_body(n_nodes, hb_ref, mp_ref, wh1h_ref, wh1m_ref, bh1_ref,
                wh2_ref, bh2_ref, out_ref):
    hb = hb_ref[...]
    m = mp_ref[:n_nodes, :] + mp_ref[n_nodes:, :]
    z = (jnp.dot(hb, wh1h_ref[...], preferred_element_type=jnp.float32)
         + jnp.dot(m, wh1m_ref[...], preferred_element_type=jnp.float32)
         + bh1_ref[...])
    z = _silu(z)
    out_ref[...] = hb + jnp.dot(z, wh2_ref[...],
                                preferred_element_type=jnp.float32) + bh2_ref[...]


def kernel(h, x, e, gamma, beta, We1, be1, We2, be2, Wi, bi, Wh1, bh1, Wh2, bh2):
    n, d = h.shape
    ne = e.shape[1]
    hh = We2.shape[0]
    epw = ne // NW
    mesh = plsc.VectorSubcoreMesh(core_axis_name="c", subcore_axis_name="s")

    hb, A, B = pl.pallas_call(
        _prep_body,
        out_shape=[jax.ShapeDtypeStruct((n, d), jnp.float32),
                   jax.ShapeDtypeStruct((n, hh), jnp.float32),
                   jax.ShapeDtypeStruct((n, hh), jnp.float32)],
    )(h, gamma.reshape(1, d), beta.reshape(1, d), We1[:d], We1[d:2 * d])

    es = e[0]
    ed = e[1]
    xc = x.T.astype(jnp.float32)
    gather = pl.kernel(
        functools.partial(_gather_body, n, epw),
        out_type=[jax.ShapeDtypeStruct((ne, hh), jnp.float32),
                  jax.ShapeDtypeStruct((ne, hh), jnp.float32),
                  jax.ShapeDtypeStruct((ne,), jnp.float32)],
        mesh=mesh,
        scratch_types=[pltpu.VMEM((n,), jnp.float32),
                       pltpu.VMEM((n,), jnp.float32),
                       pltpu.VMEM((n,), jnp.float32),
                       pltpu.VMEM((CH,), jnp.int32),
                       pltpu.VMEM((CH,), jnp.int32),
                       pltpu.VMEM((CH, hh), jnp.float32),
                       pltpu.VMEM((CH, hh), jnp.float32),
                       pltpu.VMEM((CH,), jnp.float32),
                       pltpu.SemaphoreType.DMA],
    )
    As, Bd, d2 = gather(A, B, xc, es, ed)

    R = 2000
    grid = ne // R
    wm = pl.pallas_call(
        _edge_body,
        grid=(grid,),
        in_specs=[
            pl.BlockSpec((R, hh), lambda i: (i, 0)),
            pl.BlockSpec((R, hh), lambda i: (i, 0)),
            pl.BlockSpec((R, 1), lambda i: (i, 0)),
            pl.BlockSpec((1, hh), lambda i: (0, 0)),
            pl.BlockSpec((1, hh), lambda i: (0, 0)),
            pl.BlockSpec((hh, hh), lambda i: (0, 0)),
            pl.BlockSpec((1, hh), lambda i: (0, 0)),
            pl.BlockSpec((1, hh), lambda i: (0, 0)),
            pl.BlockSpec(memory_space=pltpu.MemorySpace.SMEM),
        ],
        out_specs=pl.BlockSpec((R, hh), lambda i: (i, 0)),
        out_shape=jax.ShapeDtypeStruct((ne, hh), jnp.float32),
    )(As, Bd, d2.reshape(ne, 1), We1[2 * d:2 * d + 1], be1.reshape(1, hh),
      We2, be2.reshape(1, hh), Wi.reshape(1, hh), bi)

    zeros = jnp.zeros((n, hh), jnp.float32)
    scatter = pl.kernel(
        functools.partial(_scatter_body, n, epw),
        out_type=jax.ShapeDtypeStruct((NC * n, hh), jnp.float32),
        mesh=mesh,
        scratch_types=[pltpu.VMEM_SHARED((n, hh), jnp.float32),
                       pltpu.VMEM((CH, hh), jnp.float32),
                       pltpu.VMEM((CH,), jnp.int32),
                       pltpu.SemaphoreType.DMA],
    )
    mparts = scatter(wm, es, zeros)

    h_out = pl.pallas_call(
        functools.partial(_final_body, n),
        out_shape=jax.ShapeDtypeStruct((n, d), jnp.float32),
    )(hb, mparts, Wh1[:d], Wh1[d:], bh1.reshape(1, hh), Wh2,
      bh2.reshape(1, d))

    return (h_out, e)


# speedup vs baseline: 3.3709x; 3.3709x over previous
import functools

import jax
import jax.numpy as jnp
from jax import lax
from jax.experimental import pallas as pl
from jax.experimental.pallas import tpu as pltpu
from jax.experimental.pallas import tpu_sc as plsc

NC = 2
NS = 16
NW = NC * NS
CH = 80


def _silu(v):
    return v * jax.nn.sigmoid(v)


def _prep_body(h_ref, g_ref, b_ref, wa_ref, wb_ref, hb_ref, a_ref, bb_ref):
    h = h_ref[...]
    mean = jnp.mean(h, axis=0, keepdims=True)
    var = jnp.mean((h - mean) ** 2, axis=0, keepdims=True)
    hb = g_ref[...] * (h - mean) / jnp.sqrt(var + 1e-5) + b_ref[...]
    hb_ref[...] = hb
    a_ref[...] = jnp.dot(hb, wa_ref[...], preferred_element_type=jnp.float32)
    bb_ref[...] = jnp.dot(hb, wb_ref[...], preferred_element_type=jnp.float32)


def _gather_body(n_nodes, epw, a_hbm, b_hbm, xp_hbm, es_hbm, ed_hbm,
                 as_out, bd_out, xs_out, xd_out,
                 idx_s, idx_d, asb, bdb, xsb, xdb, sem):
    c = lax.axis_index("c")
    s = lax.axis_index("s")
    base = (s * NC + c) * epw

    def chunk(j, carry):
        off = base + j * CH
        pltpu.sync_copy(es_hbm.at[pl.ds(off, CH)], idx_s)
        pltpu.sync_copy(ed_hbm.at[pl.ds(off, CH)], idx_d)
        cp_a = pltpu.async_copy(a_hbm.at[idx_s], asb, sem)
        cp_b = pltpu.async_copy(b_hbm.at[idx_d], bdb, sem)
        cp_xs = pltpu.async_copy(xp_hbm.at[idx_s], xsb, sem)
        cp_xd = pltpu.async_copy(xp_hbm.at[idx_d], xdb, sem)
        cp_a.wait()
        cp_b.wait()
        cp_xs.wait()
        cp_xd.wait()
        pltpu.sync_copy(asb, as_out.at[pl.ds(off, CH)])
        pltpu.sync_copy(bdb, bd_out.at[pl.ds(off, CH)])
        pltpu.sync_copy(xsb, xs_out.at[pl.ds(off, CH)])
        pltpu.sync_copy(xdb, xd_out.at[pl.ds(off, CH)])
        return carry

    lax.fori_loop(0, epw // CH, chunk, 0)


def _edge_body(as_ref, bd_ref, xs_ref, xd_ref, w256_ref, be1_ref, we2_ref,
               be2_ref, wit_ref, bi_ref, out_ref):
    diff = xs_ref[...] - xd_ref[...]
    dist = jnp.sqrt(jnp.sum(diff * diff, axis=1, keepdims=True))
    pre1 = as_ref[...] + bd_ref[...] + dist * w256_ref[...] + be1_ref[...]
    u = _silu(pre1)
    v = jnp.dot(u, we2_ref[...], preferred_element_type=jnp.float32)
    v = _silu(v + be2_ref[...])
    logit = jnp.sum(v * wit_ref[...], axis=1, keepdims=True) + bi_ref[0]
    out_ref[...] = jax.nn.sigmoid(logit) * v


def _scatter_body(n_nodes, epw, wm_hbm, es_hbm, zeros_hbm, out_hbm,
                  acc, rows, idx, sem):
    c = lax.axis_index("c")
    s = lax.axis_index("s")
    npt = (n_nodes // NS) // 8 * 8
    rem = n_nodes - npt * NS
    pltpu.sync_copy(zeros_hbm.at[pl.ds(s * npt, npt)],
                    acc.at[pl.ds(s * npt, npt)])

    @pl.when(s == 0)
    def _():
        pltpu.sync_copy(zeros_hbm.at[pl.ds(npt * NS, rem)],
                        acc.at[pl.ds(npt * NS, rem)])

    plsc.subcore_barrier()
    base = (s * NC + c) * epw

    def chunk(j, carry):
        off = base + j * CH
        pltpu.sync_copy(es_hbm.at[pl.ds(off, CH)], idx)
        pltpu.sync_copy(wm_hbm.at[pl.ds(off, CH)], rows)
        pltpu.sync_copy(rows, acc.at[idx], add=True)
        return carry

    lax.fori_loop(0, epw // CH, chunk, 0)
    plsc.subcore_barrier()
    obase = c * n_nodes + s * npt
    pltpu.sync_copy(acc.at[pl.ds(s * npt, npt)], out_hbm.at[pl.ds(obase, npt)])

    @pl.when(s == 0)
    def _():
        pltpu.sync_copy(acc.at[pl.ds(npt * NS, rem)],
                        out_hbm.at[pl.ds(c * n_nodes + npt * NS, rem)])


def _final_body(n_nodes, hb_ref, mp_ref, wh1h_ref, wh1m_ref, bh1_ref,
                wh2_ref, bh2_ref, out_ref):
    hb = hb_ref[...]
    m = mp_ref[:n_nodes, :] + mp_ref[n_nodes:, :]
    z = (jnp.dot(hb, wh1h_ref[...], preferred_element_type=jnp.float32)
         + jnp.dot(m, wh1m_ref[...], preferred_element_type=jnp.float32)
         + bh1_ref[...])
    z = _silu(z)
    out_ref[...] = hb + jnp.dot(z, wh2_ref[...],
                                preferred_element_type=jnp.float32) + bh2_ref[...]


def kernel(h, x, e, gamma, beta, We1, be1, We2, be2, Wi, bi, Wh1, bh1, Wh2, bh2):
    n, d = h.shape
    ne = e.shape[1]
    hh = We2.shape[0]
    epw = ne // NW
    mesh = plsc.VectorSubcoreMesh(core_axis_name="c", subcore_axis_name="s")

    hb, A, B = pl.pallas_call(
        _prep_body,
        out_shape=[jax.ShapeDtypeStruct((n, d), jnp.float32),
                   jax.ShapeDtypeStruct((n, hh), jnp.float32),
                   jax.ShapeDtypeStruct((n, hh), jnp.float32)],
    )(h, gamma.reshape(1, d), beta.reshape(1, d), We1[:d], We1[d:2 * d])

    es = e[0]
    ed = e[1]
    xp = jnp.pad(x.astype(jnp.float32), ((0, 0), (0, 16 - x.shape[1])))
    gather = pl.kernel(
        functools.partial(_gather_body, n, epw),
        out_type=[jax.ShapeDtypeStruct((ne, hh), jnp.float32),
                  jax.ShapeDtypeStruct((ne, hh), jnp.float32),
                  jax.ShapeDtypeStruct((ne, 16), jnp.float32),
                  jax.ShapeDtypeStruct((ne, 16), jnp.float32)],
        mesh=mesh,
        scratch_types=[pltpu.VMEM((CH,), jnp.int32),
                       pltpu.VMEM((CH,), jnp.int32),
                       pltpu.VMEM((CH, hh), jnp.float32),
                       pltpu.VMEM((CH, hh), jnp.float32),
                       pltpu.VMEM((CH, 16), jnp.float32),
                       pltpu.VMEM((CH, 16), jnp.float32),
                       pltpu.SemaphoreType.DMA],
        compiler_params=pltpu.CompilerParams(use_tc_tiling_on_sc=False),
    )
    As, Bd, Xs, Xd = gather(A, B, xp, es, ed)

    R = 2000
    grid = ne // R
    wm = pl.pallas_call(
        _edge_body,
        grid=(grid,),
        in_specs=[
            pl.BlockSpec((R, hh), lambda i: (i, 0)),
            pl.BlockSpec((R, hh), lambda i: (i, 0)),
            pl.BlockSpec((R, 16), lambda i: (i, 0)),
            pl.BlockSpec((R, 16), lambda i: (i, 0)),
            pl.BlockSpec((1, hh), lambda i: (0, 0)),
            pl.BlockSpec((1, hh), lambda i: (0, 0)),
            pl.BlockSpec((hh, hh), lambda i: (0, 0)),
            pl.BlockSpec((1, hh), lambda i: (0, 0)),
            pl.BlockSpec((1, hh), lambda i: (0, 0)),
            pl.BlockSpec(memory_space=pltpu.MemorySpace.SMEM),
        ],
        out_specs=pl.BlockSpec((R, hh), lambda i: (i, 0)),
        out_shape=jax.ShapeDtypeStruct((ne, hh), jnp.float32),
    )(As, Bd, Xs, Xd, We1[2 * d:2 * d + 1], be1.reshape(1, hh),
      We2, be2.reshape(1, hh), Wi.reshape(1, hh), bi)

    zeros = jnp.zeros((n, hh), jnp.float32)
    scatter = pl.kernel(
        functools.partial(_scatter_body, n, epw),
        out_type=jax.ShapeDtypeStruct((NC * n, hh), jnp.float32),
        mesh=mesh,
        scratch_types=[pltpu.VMEM_SHARED((n, hh), jnp.float32),
                       pltpu.VMEM((CH, hh), jnp.float32),
                       pltpu.VMEM((CH,), jnp.int32),
                       pltpu.SemaphoreType.DMA],
    )
    mparts = scatter(wm, es, zeros)

    h_out = pl.pallas_call(
        functools.partial(_final_body, n),
        out_shape=jax.ShapeDtypeStruct((n, d), jnp.float32),
    )(hb, mparts, Wh1[:d], Wh1[d:], bh1.reshape(1, hh), Wh2,
      bh2.reshape(1, d))

    return (h_out, e)


# speedup vs baseline: 4.4298x; 1.3141x over previous
import functools

import jax
import jax.numpy as jnp
from jax import lax
from jax.experimental import pallas as pl
from jax.experimental.pallas import tpu as pltpu
from jax.experimental.pallas import tpu_sc as plsc

NC = 2
NS = 16
NW = NC * NS
CH = 80


def _silu(v):
    return v * jax.nn.sigmoid(v)


def _prep_body(h_ref, g_ref, b_ref, wa_ref, wb_ref, hb_ref, a_ref, bb_ref):
    h = h_ref[...]
    mean = jnp.mean(h, axis=0, keepdims=True)
    var = jnp.mean((h - mean) ** 2, axis=0, keepdims=True)
    hb = g_ref[...] * (h - mean) / jnp.sqrt(var + 1e-5) + b_ref[...]
    hb_ref[...] = hb
    a_ref[...] = jnp.dot(hb, wa_ref[...], preferred_element_type=jnp.float32)
    bb_ref[...] = jnp.dot(hb, wb_ref[...], preferred_element_type=jnp.float32)


def _row_add(dst, src, width):
    def row(r, carry):
        for g in range(width // 16):
            sl = pl.ds(g * 16, 16)
            dst[r, sl] = dst[r, sl] + src[r, sl]
        return carry

    lax.fori_loop(0, CH, row, 0)


def _gather_body(n_nodes, epw, a_hbm, b_hbm, xp_hbm, es_hbm, ed_hbm,
                 p0_out, xs_out, xd_out,
                 ids_s, ids_d, asb0, bdb0, xsb0, xdb0, asb1, bdb1, xsb1, xdb1,
                 sem_i, sem_g0, sem_g1, sem_o0, sem_o1):
    c = lax.axis_index("c")
    s = lax.axis_index("s")
    base = (s * NC + c) * epw
    cp1 = pltpu.async_copy(es_hbm.at[pl.ds(base, epw)], ids_s, sem_i)
    cp2 = pltpu.async_copy(ed_hbm.at[pl.ds(base, epw)], ids_d, sem_i)
    cp1.wait()
    cp2.wait()

    sets = ((asb0, bdb0, xsb0, xdb0, sem_g0, sem_o0),
            (asb1, bdb1, xsb1, xdb1, sem_g1, sem_o1))

    def fire(j, st):
        asb, bdb, xsb, xdb, sem_g, _ = st
        loc = j * CH
        si = ids_s.at[pl.ds(loc, CH)]
        di = ids_d.at[pl.ds(loc, CH)]
        return (pltpu.async_copy(a_hbm.at[si], asb, sem_g),
                pltpu.async_copy(b_hbm.at[di], bdb, sem_g),
                pltpu.async_copy(xp_hbm.at[si], xsb, sem_g),
                pltpu.async_copy(xp_hbm.at[di], xdb, sem_g))

    def finish(j, st, cps):
        asb, bdb, xsb, xdb, _, sem_o = st
        for cp in cps:
            cp.wait()
        _row_add(asb, bdb, asb.shape[1])
        off = base + j * CH
        return (pltpu.async_copy(asb, p0_out.at[pl.ds(off, CH)], sem_o),
                pltpu.async_copy(xsb, xs_out.at[pl.ds(off, CH)], sem_o),
                pltpu.async_copy(xdb, xd_out.at[pl.ds(off, CH)], sem_o))

    nch = epw // CH

    def pair(jj, carry):
        j0 = jj * 2
        cps0 = fire(j0, sets[0])
        cps1 = fire(j0 + 1, sets[1])
        out0 = finish(j0, sets[0], cps0)
        out1 = finish(j0 + 1, sets[1], cps1)
        for cp in out0 + out1:
            cp.wait()
        return carry

    lax.fori_loop(0, nch // 2, pair, 0)
    if nch % 2:
        j = nch - 1
        cps = fire(j, sets[0])
        outs = finish(j, sets[0], cps)
        for cp in outs:
            cp.wait()


def _edge_body(p0_ref, xs_ref, xd_ref, w256_ref, be1_ref, we2_ref,
               be2_ref, wit_ref, bi_ref, out_ref):
    diff = xs_ref[...] - xd_ref[...]
    dist = jnp.sqrt(jnp.sum(diff * diff, axis=1, keepdims=True))
    pre1 = p0_ref[...] + dist * w256_ref[...] + be1_ref[...]
    u = _silu(pre1)
    v = jnp.dot(u, we2_ref[...], preferred_element_type=jnp.float32)
    v = _silu(v + be2_ref[...])
    logit = jnp.sum(v * wit_ref[...], axis=1, keepdims=True) + bi_ref[0]
    out_ref[...] = jax.nn.sigmoid(logit) * v


def _scatter_body(n_nodes, epw, wm_hbm, es_hbm, zeros_hbm, out_hbm,
                  acc, rows0, idx0, rows1, idx1,
                  sem_l0, sem_s0, sem_l1, sem_s1):
    c = lax.axis_index("c")
    s = lax.axis_index("s")
    npt = (n_nodes // NS) // 8 * 8
    rem = n_nodes - npt * NS
    pltpu.sync_copy(zeros_hbm.at[pl.ds(s * npt, npt)],
                    acc.at[pl.ds(s * npt, npt)])

    @pl.when(s == 0)
    def _():
        pltpu.sync_copy(zeros_hbm.at[pl.ds(npt * NS, rem)],
                        acc.at[pl.ds(npt * NS, rem)])

    plsc.subcore_barrier()
    base = (s * NC + c) * epw

    sets = ((rows0, idx0, sem_l0, sem_s0), (rows1, idx1, sem_l1, sem_s1))

    def load(j, st):
        rows, idx, sem_l, _ = st
        off = base + j * CH
        return (pltpu.async_copy(wm_hbm.at[pl.ds(off, CH)], rows, sem_l),
                pltpu.async_copy(es_hbm.at[pl.ds(off, CH)], idx, sem_l))

    def scat(st, cps):
        rows, idx, _, sem_s = st
        for cp in cps:
            cp.wait()
        return pltpu.async_copy(rows, acc.at[idx], sem_s, add=True)

    nch = epw // CH

    def pair(jj, carry):
        j0 = jj * 2
        cps0 = load(j0, sets[0])
        cps1 = load(j0 + 1, sets[1])
        sc0 = scat(sets[0], cps0)
        sc1 = scat(sets[1], cps1)
        sc0.wait()
        sc1.wait()
        return carry

    lax.fori_loop(0, nch // 2, pair, 0)
    if nch % 2:
        cps = load(nch - 1, sets[0])
        scat(sets[0], cps).wait()
    plsc.subcore_barrier()
    obase = c * n_nodes + s * npt
    pltpu.sync_copy(acc.at[pl.ds(s * npt, npt)], out_hbm.at[pl.ds(obase, npt)])

    @pl.when(s == 0)
    def _():
        pltpu.sync_copy(acc.at[pl.ds(npt * NS, rem)],
                        out_hbm.at[pl.ds(c * n_nodes + npt * NS, rem)])


def _final_body(n_nodes, hb_ref, mp_ref, wh1h_ref, wh1m_ref, bh1_ref,
                wh2_ref, bh2_ref, out_ref):
    hb = hb_ref[...]
    m = mp_ref[:n_nodes, :] + mp_ref[n_nodes:, :]
    z = (jnp.dot(hb, wh1h_ref[...], preferred_element_type=jnp.float32)
         + jnp.dot(m, wh1m_ref[...], preferred_element_type=jnp.float32)
         + bh1_ref[...])
    z = _silu(z)
    out_ref[...] = hb + jnp.dot(z, wh2_ref[...],
                                preferred_element_type=jnp.float32) + bh2_ref[...]


def kernel(h, x, e, gamma, beta, We1, be1, We2, be2, Wi, bi, Wh1, bh1, Wh2, bh2):
    n, d = h.shape
    ne = e.shape[1]
    hh = We2.shape[0]
    epw = ne // NW
    mesh = plsc.VectorSubcoreMesh(core_axis_name="c", subcore_axis_name="s")

    hb, A, B = pl.pallas_call(
        _prep_body,
        out_shape=[jax.ShapeDtypeStruct((n, d), jnp.float32),
                   jax.ShapeDtypeStruct((n, hh), jnp.float32),
                   jax.ShapeDtypeStruct((n, hh), jnp.float32)],
    )(h, gamma.reshape(1, d), beta.reshape(1, d), We1[:d], We1[d:2 * d])

    es = e[0]
    ed = e[1]
    xp = jnp.pad(x.astype(jnp.float32), ((0, 0), (0, 16 - x.shape[1])))
    gather = pl.kernel(
        functools.partial(_gather_body, n, epw),
        out_type=[jax.ShapeDtypeStruct((ne, hh), jnp.float32),
                  jax.ShapeDtypeStruct((ne, 16), jnp.float32),
                  jax.ShapeDtypeStruct((ne, 16), jnp.float32)],
        mesh=mesh,
        scratch_types=[pltpu.VMEM((epw,), jnp.int32),
                       pltpu.VMEM((epw,), jnp.int32),
                       pltpu.VMEM((CH, hh), jnp.float32),
                       pltpu.VMEM((CH, hh), jnp.float32),
                       pltpu.VMEM((CH, 16), jnp.float32),
                       pltpu.VMEM((CH, 16), jnp.float32),
                       pltpu.VMEM((CH, hh), jnp.float32),
                       pltpu.VMEM((CH, hh), jnp.float32),
                       pltpu.VMEM((CH, 16), jnp.float32),
                       pltpu.VMEM((CH, 16), jnp.float32),
                       pltpu.SemaphoreType.DMA,
                       pltpu.SemaphoreType.DMA,
                       pltpu.SemaphoreType.DMA,
                       pltpu.SemaphoreType.DMA,
                       pltpu.SemaphoreType.DMA],
        compiler_params=pltpu.CompilerParams(use_tc_tiling_on_sc=False),
    )
    P0, Xs, Xd = gather(A, B, xp, es, ed)

    R = 2000
    grid = ne // R
    wm = pl.pallas_call(
        _edge_body,
        grid=(grid,),
        in_specs=[
            pl.BlockSpec((R, hh), lambda i: (i, 0)),
            pl.BlockSpec((R, 16), lambda i: (i, 0)),
            pl.BlockSpec((R, 16), lambda i: (i, 0)),
            pl.BlockSpec((1, hh), lambda i: (0, 0)),
            pl.BlockSpec((1, hh), lambda i: (0, 0)),
            pl.BlockSpec((hh, hh), lambda i: (0, 0)),
            pl.BlockSpec((1, hh), lambda i: (0, 0)),
            pl.BlockSpec((1, hh), lambda i: (0, 0)),
            pl.BlockSpec(memory_space=pltpu.MemorySpace.SMEM),
        ],
        out_specs=pl.BlockSpec((R, hh), lambda i: (i, 0)),
        out_shape=jax.ShapeDtypeStruct((ne, hh), jnp.float32),
    )(P0, Xs, Xd, We1[2 * d:2 * d + 1], be1.reshape(1, hh),
      We2, be2.reshape(1, hh), Wi.reshape(1, hh), bi)

    zeros = jnp.zeros((n, hh), jnp.float32)
    scatter = pl.kernel(
        functools.partial(_scatter_body, n, epw),
        out_type=jax.ShapeDtypeStruct((NC * n, hh), jnp.float32),
        mesh=mesh,
        scratch_types=[pltpu.VMEM_SHARED((n, hh), jnp.float32),
                       pltpu.VMEM((CH, hh), jnp.float32),
                       pltpu.VMEM((CH,), jnp.int32),
                       pltpu.VMEM((CH, hh), jnp.float32),
                       pltpu.VMEM((CH,), jnp.int32),
                       pltpu.SemaphoreType.DMA,
                       pltpu.SemaphoreType.DMA,
                       pltpu.SemaphoreType.DMA,
                       pltpu.SemaphoreType.DMA],
    )
    mparts = scatter(wm, es, zeros)

    h_out = pl.pallas_call(
        functools.partial(_final_body, n),
        out_shape=jax.ShapeDtypeStruct((n, d), jnp.float32),
    )(hb, mparts, Wh1[:d], Wh1[d:], bh1.reshape(1, hh), Wh2,
      bh2.reshape(1, d))

    return (h_out, e)
